# baseline (speedup 1.0000x reference)
import functools

import jax
import jax.numpy as jnp
from jax import lax
from jax.experimental import pallas as pl
from jax.experimental.pallas import tpu as pltpu

N = 400000
N1 = 50000
N2 = 5000
E1 = 800000
E2 = 80000
D = 64
NC = 20


def _dense_body(refs, *, has_x, has_side, side_k, has_b2, has_c, has_res, relu_last):
    i = 0
    x_ref = refs[i] if has_x else None
    i += int(has_x)
    s_ref = refs[i] if has_side else None
    i += int(has_side)
    res_ref = refs[i] if has_res else None
    i += int(has_res)
    wa_ref, ba_ref = refs[i], refs[i + 1]
    i += 2
    if has_b2:
        wb_ref, bb_ref = refs[i], refs[i + 1]
        i += 2
    if has_c:
        wc_ref, bc_ref = refs[i], refs[i + 1]
        i += 2
    o_ref = refs[i]

    acc = None
    if has_x:
        wa1 = wa_ref[0:D, :]
        acc = jnp.dot(x_ref[...], wa1, preferred_element_type=jnp.float32)
    if has_side:
        off = D if has_x else 0
        wa2 = wa_ref[off:off + side_k, :]
        t = jnp.dot(s_ref[...], wa2, preferred_element_type=jnp.float32)
        acc = t if acc is None else acc + t
    h = jax.nn.relu(acc + ba_ref[...])
    if has_b2:
        h = jax.nn.relu(jnp.dot(h, wb_ref[...], preferred_element_type=jnp.float32)
                        + bb_ref[...])
    if has_c:
        h = jnp.dot(h, wc_ref[...], preferred_element_type=jnp.float32) + bc_ref[...]
        if relu_last:
            h = jax.nn.relu(h)
    if has_res:
        h = h + res_ref[...]
    o_ref[...] = h


def _dense(x, side, Wa, ba, Wb=None, bb=None, Wc=None, bc=None, res=None,
           relu_last=False, block=1024):
    has_x = x is not None
    has_side = side is not None
    n = x.shape[0] if has_x else side.shape[0]
    side_k = side.shape[1] if has_side else 0
    nout = (Wc if Wc is not None else (Wb if Wb is not None else Wa)).shape[1]
    grid = (pl.cdiv(n, block),)

    in_specs = []
    args = []
    row_spec = lambda k: pl.BlockSpec((block, k), lambda i: (i, 0))
    whole = lambda a: pl.BlockSpec(a.shape, lambda i: (0,) * a.ndim)
    if has_x:
        args.append(x); in_specs.append(row_spec(x.shape[1]))
    if has_side:
        args.append(side); in_specs.append(row_spec(side_k))
    if res is not None:
        args.append(res); in_specs.append(row_spec(res.shape[1]))
    for w in (Wa, ba, Wb, bb, Wc, bc):
        if w is not None:
            args.append(w); in_specs.append(whole(w))

    body = functools.partial(
        _dense_body, has_x=has_x, has_side=has_side, side_k=side_k,
        has_b2=Wb is not None, has_c=Wc is not None, has_res=res is not None,
        relu_last=relu_last)

    return pl.pallas_call(
        lambda *refs: body(refs),
        grid=grid,
        in_specs=in_specs,
        out_specs=pl.BlockSpec((block, nout), lambda i: (i, 0)),
        out_shape=jax.ShapeDtypeStruct((n, nout), jnp.float32),
    )(*args)


def _gather_rows(table, idx):
    return table[idx]


def _seg_sum(data, ids, n):
    return jax.ops.segment_sum(data, ids, num_segments=n)


def _seg_max0(data, ids, n):
    agg = jax.ops.segment_max(data, ids, num_segments=n)
    return jnp.where(jnp.isfinite(agg), agg, 0.0)


def kernel(remission, points, l1_cluster_centers, l2_cluster_centers,
           l1_edges, l2_edges, l1_labels, l2_labels,
           W1a, b1a, W1b, b1b, W2ea, b2ea, W2eb, b2eb,
           W2oa, b2oa, W2ob, b2ob, W3, b3, W5, b5,
           W6ea, b6ea, W6eb, b6eb, W6oa, b6oa, W6ob, b6ob,
           W7, b7, Wc, bc):
    l1cc = l1_cluster_centers
    l2cc = l2_cluster_centers

    rel1 = points - _gather_rows(l1cc, l1_labels)
    x4 = jnp.concatenate([remission, rel1], axis=1)
    pf = _dense(None, x4, W1a, b1a, W1b, b1b)
    t1 = _seg_sum(pf, l1_labels, N1)

    src1, dst1 = l1_edges[0], l1_edges[1]
    dcc1 = _gather_rows(l1cc, src1) - _gather_rows(l1cc, dst1)

    msg = _dense(_gather_rows(t1, src1), dcc1, W2ea, b2ea, W2eb, b2eb)
    agg = _seg_max0(msg, dst1, N1)
    t2 = _dense(agg, None, W2oa, b2oa, W2ob, b2ob, res=t1)

    rel3 = l1cc - _gather_rows(l2cc, l2_labels)
    h3 = _dense(t2, rel3, W3, b3)
    t3 = _seg_max0(h3, l2_labels, N2)

    src2, dst2 = l2_edges[0], l2_edges[1]
    t4 = t3 + _seg_max0(_gather_rows(t3, src2), dst2, N2)

    t5 = _dense(None, rel3, W5, b5, res=_gather_rows(t4, l2_labels))

    msg6 = _dense(_gather_rows(t5, src1), dcc1, W6ea, b6ea, W6eb, b6eb)
    agg6 = _seg_max0(msg6, dst1, N1)
    t6 = _dense(agg6, None, W6oa, b6oa, W6ob, b6ob, res=t5)
    t6 = t6 + t2

    out = _dense(_gather_rows(t6, l1_labels), rel1, W7, b7, Wc=Wc, bc=bc)
    return out


# speedup vs baseline: 1.1227x; 1.1227x over previous
import functools

import jax
import jax.numpy as jnp
from jax import lax
from jax.experimental import pallas as pl
from jax.experimental.pallas import tpu as pltpu
from jax.experimental.pallas import tpu_sc as plsc

N = 400000
N1 = 50000
N2 = 5000
E1 = 800000
E2 = 80000
D = 64
NC = 20


def _dense_body(refs, *, has_x, has_side, side_k, has_b2, has_c, has_res, relu_last):
    i = 0
    x_ref = refs[i] if has_x else None
    i += int(has_x)
    s_ref = refs[i] if has_side else None
    i += int(has_side)
    res_ref = refs[i] if has_res else None
    i += int(has_res)
    wa_ref, ba_ref = refs[i], refs[i + 1]
    i += 2
    if has_b2:
        wb_ref, bb_ref = refs[i], refs[i + 1]
        i += 2
    if has_c:
        wc_ref, bc_ref = refs[i], refs[i + 1]
        i += 2
    o_ref = refs[i]

    acc = None
    if has_x:
        wa1 = wa_ref[0:D, :]
        acc = jnp.dot(x_ref[...], wa1, preferred_element_type=jnp.float32)
    if has_side:
        off = D if has_x else 0
        wa2 = wa_ref[off:off + side_k, :]
        t = jnp.dot(s_ref[...], wa2, preferred_element_type=jnp.float32)
        acc = t if acc is None else acc + t
    h = jax.nn.relu(acc + ba_ref[...])
    if has_b2:
        h = jax.nn.relu(jnp.dot(h, wb_ref[...], preferred_element_type=jnp.float32)
                        + bb_ref[...])
    if has_c:
        h = jnp.dot(h, wc_ref[...], preferred_element_type=jnp.float32) + bc_ref[...]
        if relu_last:
            h = jax.nn.relu(h)
    if has_res:
        h = h + res_ref[...]
    o_ref[...] = h


def _dense(x, side, Wa, ba, Wb=None, bb=None, Wc=None, bc=None, res=None,
           relu_last=False, block=1024):
    has_x = x is not None
    has_side = side is not None
    n = x.shape[0] if has_x else side.shape[0]
    side_k = side.shape[1] if has_side else 0
    nout = (Wc if Wc is not None else (Wb if Wb is not None else Wa)).shape[1]
    grid = (pl.cdiv(n, block),)

    in_specs = []
    args = []
    row_spec = lambda k: pl.BlockSpec((block, k), lambda i: (i, 0))
    whole = lambda a: pl.BlockSpec(a.shape, lambda i: (0,) * a.ndim)
    if has_x:
        args.append(x); in_specs.append(row_spec(x.shape[1]))
    if has_side:
        args.append(side); in_specs.append(row_spec(side_k))
    if res is not None:
        args.append(res); in_specs.append(row_spec(res.shape[1]))
    for w in (Wa, ba, Wb, bb, Wc, bc):
        if w is not None:
            args.append(w); in_specs.append(whole(w))

    body = functools.partial(
        _dense_body, has_x=has_x, has_side=has_side, side_k=side_k,
        has_b2=Wb is not None, has_c=Wc is not None, has_res=res is not None,
        relu_last=relu_last)

    return pl.pallas_call(
        lambda *refs: body(refs),
        grid=grid,
        in_specs=in_specs,
        out_specs=pl.BlockSpec((block, nout), lambda i: (i, 0)),
        out_shape=jax.ShapeDtypeStruct((n, nout), jnp.float32),
    )(*args)


_NCORE, _NSUB, _NW = 2, 16, 32
_SC_MESH = plsc.VectorSubcoreMesh(core_axis_name="c", subcore_axis_name="s")


@functools.lru_cache(maxsize=None)
def _sc_gather_call(R, V, Dt, KB):
    RPW = -(-(-(-R // _NW)) // KB) * KB
    NBLK = RPW // KB

    @functools.partial(
        pl.kernel,
        out_type=jax.ShapeDtypeStruct((R * 128, Dt), jnp.float32),
        mesh=_SC_MESH,
        compiler_params=pltpu.CompilerParams(use_tc_tiling_on_sc=False),
        scratch_types=[
            pltpu.VMEM((KB * 128,), jnp.int32),
            pltpu.VMEM((KB * 128, Dt), jnp.float32),
            pltpu.SemaphoreType.DMA,
        ],
    )
    def gk(table_h, idx_h, out_h, idx_v, rows_v, gsem):
        wid = lax.axis_index("s") * _NCORE + lax.axis_index("c")
        base = jnp.minimum(wid * RPW, R - RPW)

        def blk(b, _):
            rb = jnp.minimum(base + b * KB, R - KB)
            pltpu.sync_copy(idx_h.at[pl.ds(rb * 128, KB * 128)], idx_v)
            copies = [pltpu.async_copy(table_h.at[idx_v.at[pl.ds(j * 128, 128)]],
                                       rows_v.at[pl.ds(j * 128, 128)], gsem)
                      for j in range(KB)]
            for c in copies:
                c.wait()
            pltpu.sync_copy(rows_v, out_h.at[pl.ds(rb * 128, KB * 128)])
            return ()

        lax.fori_loop(0, NBLK, blk, ())

    return gk


def _gather_rows_sc(table, idx, KB=8):
    B = idx.shape[0]
    gk = _sc_gather_call(B // 128, table.shape[0], table.shape[1], KB)
    return gk(table, idx.astype(jnp.int32))


def _gather_rows(table, idx):
    return table[idx]


def _seg_sum(data, ids, n):
    return jax.ops.segment_sum(data, ids, num_segments=n)


def _seg_max0(data, ids, n):
    agg = jax.ops.segment_max(data, ids, num_segments=n)
    return jnp.where(jnp.isfinite(agg), agg, 0.0)


def kernel(remission, points, l1_cluster_centers, l2_cluster_centers,
           l1_edges, l2_edges, l1_labels, l2_labels,
           W1a, b1a, W1b, b1b, W2ea, b2ea, W2eb, b2eb,
           W2oa, b2oa, W2ob, b2ob, W3, b3, W5, b5,
           W6ea, b6ea, W6eb, b6eb, W6oa, b6oa, W6ob, b6ob,
           W7, b7, Wc, bc):
    l1cc = l1_cluster_centers
    l2cc = l2_cluster_centers

    rel1 = points - _gather_rows(l1cc, l1_labels)
    x4 = jnp.concatenate([remission, rel1], axis=1)
    pf = _dense(None, x4, W1a, b1a, W1b, b1b)
    t1 = _seg_sum(pf, l1_labels, N1)

    src1, dst1 = l1_edges[0], l1_edges[1]
    dcc1 = _gather_rows(l1cc, src1) - _gather_rows(l1cc, dst1)

    msg = _dense(_gather_rows_sc(t1, src1), dcc1, W2ea, b2ea, W2eb, b2eb)
    agg = _seg_max0(msg, dst1, N1)
    t2 = _dense(agg, None, W2oa, b2oa, W2ob, b2ob, res=t1)

    rel3 = l1cc - _gather_rows(l2cc, l2_labels)
    h3 = _dense(t2, rel3, W3, b3)
    t3 = _seg_max0(h3, l2_labels, N2)

    src2, dst2 = l2_edges[0], l2_edges[1]
    t4 = t3 + _seg_max0(_gather_rows_sc(t3, src2), dst2, N2)

    t5 = _dense(None, rel3, W5, b5, res=_gather_rows(t4, l2_labels))

    msg6 = _dense(_gather_rows_sc(t5, src1), dcc1, W6ea, b6ea, W6eb, b6eb)
    agg6 = _seg_max0(msg6, dst1, N1)
    t6 = _dense(agg6, None, W6oa, b6oa, W6ob, b6ob, res=t5)
    t6 = t6 + t2

    out = _dense(_gather_rows_sc(t6, l1_labels), rel1, W7, b7, Wc=Wc, bc=bc)
    return out


# speedup vs baseline: 1.4513x; 1.2927x over previous
import functools

import jax
import jax.numpy as jnp
from jax import lax
from jax.experimental import pallas as pl
from jax.experimental.pallas import tpu as pltpu
from jax.experimental.pallas import tpu_sc as plsc

N = 400000
N1 = 50000
N2 = 5000
E1 = 800000
E2 = 80000
D = 64
NC = 20


def _dense_body(refs, *, has_x, has_side, side_k, has_b2, has_c, has_res, relu_last):
    i = 0
    x_ref = refs[i] if has_x else None
    i += int(has_x)
    s_ref = refs[i] if has_side else None
    i += int(has_side)
    res_ref = refs[i] if has_res else None
    i += int(has_res)
    wa_ref, ba_ref = refs[i], refs[i + 1]
    i += 2
    if has_b2:
        wb_ref, bb_ref = refs[i], refs[i + 1]
        i += 2
    if has_c:
        wc_ref, bc_ref = refs[i], refs[i + 1]
        i += 2
    o_ref = refs[i]

    acc = None
    if has_x:
        wa1 = wa_ref[0:D, :]
        acc = jnp.dot(x_ref[...], wa1, preferred_element_type=jnp.float32)
    if has_side:
        off = D if has_x else 0
        wa2 = wa_ref[off:off + side_k, :]
        t = jnp.dot(s_ref[...], wa2, preferred_element_type=jnp.float32)
        acc = t if acc is None else acc + t
    h = jax.nn.relu(acc + ba_ref[...])
    if has_b2:
        h = jax.nn.relu(jnp.dot(h, wb_ref[...], preferred_element_type=jnp.float32)
                        + bb_ref[...])
    if has_c:
        h = jnp.dot(h, wc_ref[...], preferred_element_type=jnp.float32) + bc_ref[...]
        if relu_last:
            h = jax.nn.relu(h)
    if has_res:
        h = h + res_ref[...]
    o_ref[...] = h


def _dense(x, side, Wa, ba, Wb=None, bb=None, Wc=None, bc=None, res=None,
           relu_last=False, block=1024):
    has_x = x is not None
    has_side = side is not None
    n = x.shape[0] if has_x else side.shape[0]
    side_k = side.shape[1] if has_side else 0
    nout = (Wc if Wc is not None else (Wb if Wb is not None else Wa)).shape[1]
    grid = (pl.cdiv(n, block),)

    in_specs = []
    args = []
    row_spec = lambda k: pl.BlockSpec((block, k), lambda i: (i, 0))
    whole = lambda a: pl.BlockSpec(a.shape, lambda i: (0,) * a.ndim)
    if has_x:
        args.append(x); in_specs.append(row_spec(x.shape[1]))
    if has_side:
        args.append(side); in_specs.append(row_spec(side_k))
    if res is not None:
        args.append(res); in_specs.append(row_spec(res.shape[1]))
    for w in (Wa, ba, Wb, bb, Wc, bc):
        if w is not None:
            args.append(w); in_specs.append(whole(w))

    body = functools.partial(
        _dense_body, has_x=has_x, has_side=has_side, side_k=side_k,
        has_b2=Wb is not None, has_c=Wc is not None, has_res=res is not None,
        relu_last=relu_last)

    return pl.pallas_call(
        lambda *refs: body(refs),
        grid=grid,
        in_specs=in_specs,
        out_specs=pl.BlockSpec((block, nout), lambda i: (i, 0)),
        out_shape=jax.ShapeDtypeStruct((n, nout), jnp.float32),
    )(*args)


_NCORE, _NSUB, _NW = 2, 16, 32
_SC_MESH = plsc.VectorSubcoreMesh(core_axis_name="c", subcore_axis_name="s")


@functools.lru_cache(maxsize=None)
def _sc_gather_call(R, V, Dt, KB):
    RPW = -(-(-(-R // _NW)) // KB) * KB
    NBLK = RPW // KB

    @functools.partial(
        pl.kernel,
        out_type=jax.ShapeDtypeStruct((R * 128, Dt), jnp.float32),
        mesh=_SC_MESH,
        compiler_params=pltpu.CompilerParams(use_tc_tiling_on_sc=False),
        scratch_types=[
            pltpu.VMEM((KB * 128,), jnp.int32),
            pltpu.VMEM((KB * 128, Dt), jnp.float32),
            pltpu.SemaphoreType.DMA,
        ],
    )
    def gk(table_h, idx_h, out_h, idx_v, rows_v, gsem):
        wid = lax.axis_index("s") * _NCORE + lax.axis_index("c")
        base = jnp.minimum(wid * RPW, R - RPW)

        def blk(b, _):
            rb = jnp.minimum(base + b * KB, R - KB)
            pltpu.sync_copy(idx_h.at[pl.ds(rb * 128, KB * 128)], idx_v)
            copies = [pltpu.async_copy(table_h.at[idx_v.at[pl.ds(j * 128, 128)]],
                                       rows_v.at[pl.ds(j * 128, 128)], gsem)
                      for j in range(KB)]
            for c in copies:
                c.wait()
            pltpu.sync_copy(rows_v, out_h.at[pl.ds(rb * 128, KB * 128)])
            return ()

        lax.fori_loop(0, NBLK, blk, ())

    return gk


def _gather_rows_sc(table, idx, KB=8):
    B = idx.shape[0]
    Bp = -(-B // 128) * 128
    idx_p = idx if Bp == B else jnp.pad(idx, (0, Bp - B))
    gk = _sc_gather_call(Bp // 128, table.shape[0], table.shape[1], KB)
    out = gk(table, idx_p.astype(jnp.int32))
    return out if Bp == B else out[:B]


def _pad_cols(a, w):
    return jnp.pad(a, ((0, 0), (0, w - a.shape[1])))


def _seg_sum(data, ids, n):
    return jax.ops.segment_sum(data, ids, num_segments=n)


def _seg_max0(data, ids, n):
    agg = jax.ops.segment_max(data, ids, num_segments=n)
    return jnp.where(jnp.isfinite(agg), agg, 0.0)


def kernel(remission, points, l1_cluster_centers, l2_cluster_centers,
           l1_edges, l2_edges, l1_labels, l2_labels,
           W1a, b1a, W1b, b1b, W2ea, b2ea, W2eb, b2eb,
           W2oa, b2oa, W2ob, b2ob, W3, b3, W5, b5,
           W6ea, b6ea, W6eb, b6eb, W6oa, b6oa, W6ob, b6ob,
           W7, b7, Wc, bc):
    l1cc = l1_cluster_centers
    l2cc = l2_cluster_centers
    l1cc8 = _pad_cols(l1cc, 8)
    l2cc8 = _pad_cols(l2cc, 8)

    rel1 = points - _gather_rows_sc(l1cc8, l1_labels)[:, :3]
    x4 = jnp.concatenate([remission, rel1], axis=1)
    pf = _dense(None, x4, W1a, b1a, W1b, b1b)
    t1 = _seg_sum(pf, l1_labels, N1)

    src1, dst1 = l1_edges[0], l1_edges[1]
    dcc1 = (_gather_rows_sc(l1cc8, src1) - _gather_rows_sc(l1cc8, dst1))[:, :3]

    msg = _dense(_gather_rows_sc(t1, src1), dcc1, W2ea, b2ea, W2eb, b2eb)
    agg = _seg_max0(msg, dst1, N1)
    t2 = _dense(agg, None, W2oa, b2oa, W2ob, b2ob, res=t1)

    rel3 = l1cc - _gather_rows_sc(l2cc8, l2_labels)[:, :3]
    h3 = _dense(t2, rel3, W3, b3)
    t3 = _seg_max0(h3, l2_labels, N2)

    src2, dst2 = l2_edges[0], l2_edges[1]
    t4 = t3 + _seg_max0(_gather_rows_sc(t3, src2), dst2, N2)

    t5 = _dense(None, rel3, W5, b5, res=_gather_rows_sc(t4, l2_labels))

    msg6 = _dense(_gather_rows_sc(t5, src1), dcc1, W6ea, b6ea, W6eb, b6eb)
    agg6 = _seg_max0(msg6, dst1, N1)
    t6 = _dense(agg6, None, W6oa, b6oa, W6ob, b6ob, res=t5)
    t6 = t6 + t2

    out = _dense(_gather_rows_sc(t6, l1_labels), rel1, W7, b7, Wc=Wc, bc=bc)
    return out


# speedup vs baseline: 1.5570x; 1.0728x over previous
import functools

import jax
import jax.numpy as jnp
from jax import lax
from jax.experimental import pallas as pl
from jax.experimental.pallas import tpu as pltpu
from jax.experimental.pallas import tpu_sc as plsc

N = 400000
N1 = 50000
N2 = 5000
E1 = 800000
E2 = 80000
D = 64
NC = 20


def _dense_body(refs, *, has_x, has_side, side_k, has_b2, has_c, has_res, relu_last):
    i = 0
    x_ref = refs[i] if has_x else None
    i += int(has_x)
    s_ref = refs[i] if has_side else None
    i += int(has_side)
    res_ref = refs[i] if has_res else None
    i += int(has_res)
    wa_ref, ba_ref = refs[i], refs[i + 1]
    i += 2
    if has_b2:
        wb_ref, bb_ref = refs[i], refs[i + 1]
        i += 2
    if has_c:
        wc_ref, bc_ref = refs[i], refs[i + 1]
        i += 2
    o_ref = refs[i]

    acc = None
    if has_x:
        wa1 = wa_ref[0:D, :]
        acc = jnp.dot(x_ref[...], wa1, preferred_element_type=jnp.float32)
    if has_side:
        off = D if has_x else 0
        wa2 = wa_ref[off:off + side_k, :]
        t = jnp.dot(s_ref[...], wa2, preferred_element_type=jnp.float32)
        acc = t if acc is None else acc + t
    h = jax.nn.relu(acc + ba_ref[...])
    if has_b2:
        h = jax.nn.relu(jnp.dot(h, wb_ref[...], preferred_element_type=jnp.float32)
                        + bb_ref[...])
    if has_c:
        h = jnp.dot(h, wc_ref[...], preferred_element_type=jnp.float32) + bc_ref[...]
        if relu_last:
            h = jax.nn.relu(h)
    if has_res:
        h = h + res_ref[...]
    o_ref[...] = h


def _dense(x, side, Wa, ba, Wb=None, bb=None, Wc=None, bc=None, res=None,
           relu_last=False, block=2048):
    has_x = x is not None
    has_side = side is not None
    n = x.shape[0] if has_x else side.shape[0]
    side_k = side.shape[1] if has_side else 0
    nout = (Wc if Wc is not None else (Wb if Wb is not None else Wa)).shape[1]
    grid = (pl.cdiv(n, block),)

    in_specs = []
    args = []
    row_spec = lambda k: pl.BlockSpec((block, k), lambda i: (i, 0))
    whole = lambda a: pl.BlockSpec(a.shape, lambda i: (0,) * a.ndim)
    if has_x:
        args.append(x); in_specs.append(row_spec(x.shape[1]))
    if has_side:
        args.append(side); in_specs.append(row_spec(side_k))
    if res is not None:
        args.append(res); in_specs.append(row_spec(res.shape[1]))
    for w in (Wa, ba, Wb, bb, Wc, bc):
        if w is not None:
            args.append(w); in_specs.append(whole(w))

    body = functools.partial(
        _dense_body, has_x=has_x, has_side=has_side, side_k=side_k,
        has_b2=Wb is not None, has_c=Wc is not None, has_res=res is not None,
        relu_last=relu_last)

    return pl.pallas_call(
        lambda *refs: body(refs),
        grid=grid,
        in_specs=in_specs,
        out_specs=pl.BlockSpec((block, nout), lambda i: (i, 0)),
        out_shape=jax.ShapeDtypeStruct((n, nout), jnp.float32),
    )(*args)


_NCORE, _NSUB, _NW = 2, 16, 32
_SC_MESH = plsc.VectorSubcoreMesh(core_axis_name="c", subcore_axis_name="s")


@functools.lru_cache(maxsize=None)
def _sc_gather_call(R, V, Dt, KB):
    RPW = -(-(-(-R // _NW)) // KB) * KB
    NBLK = RPW // KB

    @functools.partial(
        pl.kernel,
        out_type=jax.ShapeDtypeStruct((R * 128, Dt), jnp.float32),
        mesh=_SC_MESH,
        compiler_params=pltpu.CompilerParams(use_tc_tiling_on_sc=False),
        scratch_types=[
            pltpu.VMEM((KB * 128,), jnp.int32),
            pltpu.VMEM((KB * 128, Dt), jnp.float32),
            pltpu.SemaphoreType.DMA,
        ],
    )
    def gk(table_h, idx_h, out_h, idx_v, rows_v, gsem):
        wid = lax.axis_index("s") * _NCORE + lax.axis_index("c")
        base = jnp.minimum(wid * RPW, R - RPW)

        def blk(b, _):
            rb = jnp.minimum(base + b * KB, R - KB)
            pltpu.sync_copy(idx_h.at[pl.ds(rb * 128, KB * 128)], idx_v)
            copies = [pltpu.async_copy(table_h.at[idx_v.at[pl.ds(j * 128, 128)]],
                                       rows_v.at[pl.ds(j * 128, 128)], gsem)
                      for j in range(KB)]
            for c in copies:
                c.wait()
            pltpu.sync_copy(rows_v, out_h.at[pl.ds(rb * 128, KB * 128)])
            return ()

        lax.fori_loop(0, NBLK, blk, ())

    return gk


def _gather_rows_sc(table, idx, KB=8):
    B = idx.shape[0]
    Bp = -(-B // 128) * 128
    idx_p = idx if Bp == B else jnp.pad(idx, (0, Bp - B))
    gk = _sc_gather_call(Bp // 128, table.shape[0], table.shape[1], KB)
    out = gk(table, idx_p.astype(jnp.int32))
    return out if Bp == B else out[:B]


def _pad_cols(a, w):
    return jnp.pad(a, ((0, 0), (0, w - a.shape[1])))


def _seg_sum(data, ids, n):
    return jax.ops.segment_sum(data, ids, num_segments=n)


def _seg_max0(data, ids, n):
    agg = jax.ops.segment_max(data, ids, num_segments=n)
    return jnp.where(jnp.isfinite(agg), agg, 0.0)


def kernel(remission, points, l1_cluster_centers, l2_cluster_centers,
           l1_edges, l2_edges, l1_labels, l2_labels,
           W1a, b1a, W1b, b1b, W2ea, b2ea, W2eb, b2eb,
           W2oa, b2oa, W2ob, b2ob, W3, b3, W5, b5,
           W6ea, b6ea, W6eb, b6eb, W6oa, b6oa, W6ob, b6ob,
           W7, b7, Wc, bc):
    l1cc = l1_cluster_centers
    l2cc = l2_cluster_centers
    l1cc8 = _pad_cols(l1cc, 8)
    l2cc8 = _pad_cols(l2cc, 8)

    rel1 = points - _gather_rows_sc(l1cc8, l1_labels)[:, :3]
    x4 = jnp.concatenate([remission, rel1], axis=1)
    pf = _dense(None, x4, W1a, b1a, W1b, b1b)
    t1 = _seg_sum(pf, l1_labels, N1)

    src1, dst1 = l1_edges[0], l1_edges[1]
    dcc1 = (_gather_rows_sc(l1cc8, src1) - _gather_rows_sc(l1cc8, dst1))[:, :3]

    msg = _dense(_gather_rows_sc(t1, src1), dcc1, W2ea, b2ea, W2eb, b2eb)
    agg = _seg_max0(msg, dst1, N1)
    t2 = _dense(agg, None, W2oa, b2oa, W2ob, b2ob, res=t1)

    rel3 = l1cc - _gather_rows_sc(l2cc8, l2_labels)[:, :3]
    h3 = _dense(t2, rel3, W3, b3)
    t3 = _seg_max0(h3, l2_labels, N2)

    src2, dst2 = l2_edges[0], l2_edges[1]
    t4 = t3 + _seg_max0(_gather_rows_sc(t3, src2), dst2, N2)

    t5 = _dense(None, rel3, W5, b5, res=_gather_rows_sc(t4, l2_labels))

    msg6 = _dense(_gather_rows_sc(t5, src1), dcc1, W6ea, b6ea, W6eb, b6eb)
    agg6 = _seg_max0(msg6, dst1, N1)
    t6 = _dense(agg6, None, W6oa, b6oa, W6ob, b6ob, res=t5)
    t6 = t6 + t2

    out = _dense(_gather_rows_sc(t6, l1_labels), rel1, W7, b7, Wc=Wc, bc=bc)
    return out
